# Initial kernel scaffold; baseline (speedup 1.0000x reference)
import jax
import jax.numpy as jnp
from jax.experimental import pallas as pl


def kernel(x, W1, b1, W2, b2):
    raise NotImplementedError("write your pallas kernel here")


# speedup vs baseline: 15.2993x; 15.2993x over previous
import jax
import jax.numpy as jnp
from jax import lax
from jax.experimental import pallas as pl
from jax.experimental.pallas import tpu as pltpu

_B, _C, _L = 4, 768, 8192
_H = 2048
_KEEP = 64
_LB1 = 512
_CH2 = 2048
_LB3 = 1024
_NEG = -3.0e38
_PREC = lax.Precision.HIGHEST


def _p1_body(x_ref, w1_ref, b1_ref, out_ref):
    xb = x_ref[0].astype(jnp.bfloat16)
    hb = lax.dot_general(w1_ref[...].astype(jnp.bfloat16), xb,
                         (((1,), (0,)), ((), ())),
                         preferred_element_type=jnp.float32)
    hb = hb + b1_ref[...]
    out_ref[0, 0, :] = jnp.max(hb, axis=0)


def _p2_body(cm_ref, x_ref, w1_ref, b1_ref, v_ref, hh_ref, ll_ref,
             hgT_ref, xgT_ref, cols_ref):
    iota_l = lax.broadcasted_iota(jnp.int32, (1, _L), 1)
    iota_kr = lax.broadcasted_iota(jnp.int32, (1, _KEEP), 1)
    iota_kc = lax.broadcasted_iota(jnp.int32, (_KEEP, 1), 0)
    iota_h = lax.broadcasted_iota(jnp.int32, (1, _H), 1)
    c = pl.program_id(1)

    @pl.when(c == 0)
    def _():
        def sel_body(k, carry):
            cm, cols = carry
            m = jnp.max(cm)
            j = jnp.min(jnp.where(cm == m, iota_l, _L))
            cm = jnp.where(iota_l == j, _NEG, cm)
            cols = jnp.where(iota_kc == k, j, cols)
            return cm, cols

        cm0 = cm_ref[0]
        cols0 = jnp.zeros((_KEEP, 1), jnp.int32)
        _, cols = lax.fori_loop(0, _KEEP, sel_body, (cm0, cols0))
        cols_ref[...] = cols
        xgT_ref[...] = jnp.zeros((_KEEP, _C), jnp.float32)

    cols = cols_ref[...]
    selm = (c * _CH2 + lax.broadcasted_iota(jnp.int32, (_KEEP, _CH2), 1)
            == cols).astype(jnp.bfloat16)
    xgT_ref[...] += lax.dot_general(selm, x_ref[0].astype(jnp.bfloat16),
                                    (((1,), (1,)), ((), ())),
                                    preferred_element_type=jnp.float32)

    @pl.when(c == _L // _CH2 - 1)
    def _():
        hgT = lax.dot_general(xgT_ref[...].astype(jnp.bfloat16),
                              w1_ref[...].astype(jnp.bfloat16),
                              (((1,), (1,)), ((), ())),
                              preferred_element_type=jnp.float32)
        hgT = hgT + b1_ref[...]
        hgT_ref[...] = hgT
        rmax0 = jnp.max(hgT, axis=1, keepdims=True)

        def ext_body(k, carry):
            rmax, v, hh, ll = carry
            m = jnp.max(rmax)
            j = jnp.min(jnp.where(rmax == m, iota_kc, _KEEP))
            row = hgT_ref[pl.ds(j, 1), :]
            i = jnp.min(jnp.where(row == m, iota_h, _H))
            masked = jnp.where(iota_h == i, _NEG, row)
            hgT_ref[pl.ds(j, 1), :] = masked
            nm = jnp.max(masked)
            rmax = jnp.where(iota_kc == j, nm, rmax)
            v = jnp.where(iota_kr == k, m, v)
            hh = jnp.where(iota_kr == k, i, hh)
            lcol = jnp.min(jnp.where(iota_kc == j, cols, _L))
            ll = jnp.where(iota_kr == k, lcol, ll)
            return rmax, v, hh, ll

        v0 = jnp.zeros((1, _KEEP), jnp.float32)
        z0 = jnp.zeros((1, _KEEP), jnp.int32)
        _, v, hh, ll = lax.fori_loop(0, _KEEP, ext_body, (rmax0, v0, z0, z0))
        v_ref[0] = v
        hh_ref[0] = hh
        ll_ref[0] = ll


def _p3_body(v_ref, hh_ref, ll_ref, w2_ref, b2_ref, sp_ref, y_ref,
             otv_ref, wgv_ref):
    lb = pl.program_id(1)

    @pl.when(lb == 0)
    def _():
        ot = (lax.broadcasted_iota(jnp.int32, (_H, _KEEP), 0) == hh_ref[0]
              ).astype(jnp.float32)
        otv = ot * v_ref[0]
        otv_ref[...] = otv
        wgv_ref[...] = lax.dot_general(
            w2_ref[...], otv, (((1,), (0,)), ((), ())),
            preferred_element_type=jnp.float32, precision=_PREC)

    l0 = lb * _LB3
    pt = (l0 + lax.broadcasted_iota(jnp.int32, (_LB3, _KEEP), 0) == ll_ref[0]
          ).astype(jnp.float32)
    sp_ref[0] = lax.dot_general(otv_ref[...], pt, (((1,), (1,)), ((), ())),
                                preferred_element_type=jnp.float32,
                                precision=_PREC)
    y_ref[0] = lax.dot_general(wgv_ref[...], pt, (((1,), (1,)), ((), ())),
                               preferred_element_type=jnp.float32,
                               precision=_PREC) + b2_ref[...]


def kernel(x, W1, b1, W2, b2):
    b1c = b1.reshape(_H, 1)
    b1r = b1.reshape(1, _H)
    b2c = b2.reshape(_C, 1)

    colmax = pl.pallas_call(
        _p1_body,
        grid=(_B, _L // _LB1),
        in_specs=[
            pl.BlockSpec((1, _C, _LB1), lambda b, l: (b, 0, l)),
            pl.BlockSpec((_H, _C), lambda b, l: (0, 0)),
            pl.BlockSpec((_H, 1), lambda b, l: (0, 0)),
        ],
        out_specs=pl.BlockSpec((1, 1, _LB1), lambda b, l: (b, 0, l)),
        out_shape=jax.ShapeDtypeStruct((_B, 1, _L), jnp.float32),
    )(x, W1, b1c)

    v, hh, ll = pl.pallas_call(
        _p2_body,
        grid=(_B, _L // _CH2),
        in_specs=[
            pl.BlockSpec((1, 1, _L), lambda b, c: (b, 0, 0)),
            pl.BlockSpec((1, _C, _CH2), lambda b, c: (b, 0, c)),
            pl.BlockSpec((_H, _C), lambda b, c: (0, 0)),
            pl.BlockSpec((1, _H), lambda b, c: (0, 0)),
        ],
        out_specs=[
            pl.BlockSpec((1, 1, _KEEP), lambda b, c: (b, 0, 0)),
            pl.BlockSpec((1, 1, _KEEP), lambda b, c: (b, 0, 0)),
            pl.BlockSpec((1, 1, _KEEP), lambda b, c: (b, 0, 0)),
        ],
        out_shape=[
            jax.ShapeDtypeStruct((_B, 1, _KEEP), jnp.float32),
            jax.ShapeDtypeStruct((_B, 1, _KEEP), jnp.int32),
            jax.ShapeDtypeStruct((_B, 1, _KEEP), jnp.int32),
        ],
        scratch_shapes=[pltpu.VMEM((_KEEP, _H), jnp.float32),
                        pltpu.VMEM((_KEEP, _C), jnp.float32),
                        pltpu.VMEM((_KEEP, 1), jnp.int32)],
    )(colmax, x, W1, b1r)

    sparse, y = pl.pallas_call(
        _p3_body,
        grid=(_B, _L // _LB3),
        in_specs=[
            pl.BlockSpec((1, 1, _KEEP), lambda b, l: (b, 0, 0)),
            pl.BlockSpec((1, 1, _KEEP), lambda b, l: (b, 0, 0)),
            pl.BlockSpec((1, 1, _KEEP), lambda b, l: (b, 0, 0)),
            pl.BlockSpec((_C, _H), lambda b, l: (0, 0)),
            pl.BlockSpec((_C, 1), lambda b, l: (0, 0)),
        ],
        out_specs=[
            pl.BlockSpec((1, _H, _LB3), lambda b, l: (b, 0, l)),
            pl.BlockSpec((1, _C, _LB3), lambda b, l: (b, 0, l)),
        ],
        out_shape=[
            jax.ShapeDtypeStruct((_B, _H, _L), jnp.float32),
            jax.ShapeDtypeStruct((_B, _C, _L), jnp.float32),
        ],
        scratch_shapes=[pltpu.VMEM((_H, _KEEP), jnp.float32),
                        pltpu.VMEM((_C, _KEEP), jnp.float32)],
    )(v, hh, ll, W2, b2c)

    return (y, sparse)


# speedup vs baseline: 19.4899x; 1.2739x over previous
import jax
import jax.numpy as jnp
from jax import lax
from jax.experimental import pallas as pl
from jax.experimental.pallas import tpu as pltpu

_B, _C, _L = 4, 768, 8192
_H = 2048
_KEEP = 64
_LB1 = 512
_CH2 = 2048
_LB3 = 1024
_NEG = -3.0e38
_PREC = lax.Precision.HIGHEST


def _p1_body(x_ref, w1_ref, b1_ref, out_ref):
    hb = lax.dot_general(w1_ref[...], x_ref[0], (((1,), (0,)), ((), ())),
                         preferred_element_type=jnp.float32)
    hb = hb + b1_ref[...]
    out_ref[0, 0, :] = jnp.max(hb, axis=0)


def _p2_body(cm_ref, x_ref, w1_ref, b1_ref, v_ref, hh_ref, ll_ref,
             hgT_ref, xgT_ref, cols_ref):
    iota_l = lax.broadcasted_iota(jnp.int32, (1, _L), 1)
    iota_kr = lax.broadcasted_iota(jnp.int32, (1, _KEEP), 1)
    iota_kc = lax.broadcasted_iota(jnp.int32, (_KEEP, 1), 0)
    iota_h = lax.broadcasted_iota(jnp.int32, (1, _H), 1)
    c = pl.program_id(1)

    @pl.when(c == 0)
    def _():
        def sel_body(k, carry):
            cm, cols = carry
            m = jnp.max(cm)
            j = jnp.min(jnp.where(cm == m, iota_l, _L))
            cm = jnp.where(iota_l == j, _NEG, cm)
            cols = jnp.where(iota_kc == k, j, cols)
            return cm, cols

        cm0 = cm_ref[0]
        cols0 = jnp.zeros((_KEEP, 1), jnp.int32)
        _, cols = lax.fori_loop(0, _KEEP, sel_body, (cm0, cols0))
        cols_ref[...] = cols
        xgT_ref[...] = jnp.zeros((_KEEP, _C), jnp.float32)

    cols = cols_ref[...]
    selm = (c * _CH2 + lax.broadcasted_iota(jnp.int32, (_KEEP, _CH2), 1)
            == cols).astype(jnp.bfloat16)
    xgT_ref[...] += lax.dot_general(selm, x_ref[0], (((1,), (1,)), ((), ())),
                                    preferred_element_type=jnp.float32)

    @pl.when(c == _L // _CH2 - 1)
    def _():
        hgT = lax.dot_general(xgT_ref[...].astype(jnp.bfloat16), w1_ref[...],
                              (((1,), (1,)), ((), ())),
                              preferred_element_type=jnp.float32)
        hgT = hgT + b1_ref[...]
        hgT_ref[...] = hgT
        rmax0 = jnp.max(hgT, axis=1, keepdims=True)

        def ext_body(k, carry):
            rmax, v, hh, ll = carry
            m = jnp.max(rmax)
            j = jnp.min(jnp.where(rmax == m, iota_kc, _KEEP))
            row = hgT_ref[pl.ds(j, 1), :]
            i = jnp.min(jnp.where(row == m, iota_h, _H))
            masked = jnp.where(iota_h == i, _NEG, row)
            hgT_ref[pl.ds(j, 1), :] = masked
            nm = jnp.max(masked)
            rmax = jnp.where(iota_kc == j, nm, rmax)
            v = jnp.where(iota_kr == k, m, v)
            hh = jnp.where(iota_kr == k, i, hh)
            lcol = jnp.min(jnp.where(iota_kc == j, cols, _L))
            ll = jnp.where(iota_kr == k, lcol, ll)
            return rmax, v, hh, ll

        v0 = jnp.zeros((1, _KEEP), jnp.float32)
        z0 = jnp.zeros((1, _KEEP), jnp.int32)
        _, v, hh, ll = lax.fori_loop(0, _KEEP, ext_body, (rmax0, v0, z0, z0))
        v_ref[0] = v
        hh_ref[0] = hh
        ll_ref[0] = ll


def _p3_body(v_ref, hh_ref, ll_ref, w2_ref, b2_ref, sp_ref, y_ref,
             ot0_ref, ot1_ref, yg0_ref, yg1_ref):
    lb = pl.program_id(1)

    @pl.when(lb == 0)
    def _():
        vv = v_ref[0]
        v0 = vv.astype(jnp.bfloat16).astype(jnp.float32)
        v1 = (vv - v0).astype(jnp.bfloat16).astype(jnp.float32)
        ot = (lax.broadcasted_iota(jnp.int32, (_H, _KEEP), 0) == hh_ref[0]
              ).astype(jnp.float32)
        ot0_ref[...] = (ot * v0).astype(jnp.bfloat16)
        ot1_ref[...] = (ot * v1).astype(jnp.bfloat16)
        wg = lax.dot_general(w2_ref[...], ot0_ref[...],
                             (((1,), (0,)), ((), ())),
                             preferred_element_type=jnp.float32)
        yg0 = wg.astype(jnp.bfloat16)
        yg0_ref[...] = yg0
        yg1_ref[...] = (wg - yg0.astype(jnp.float32)).astype(jnp.bfloat16)

    l0 = lb * _LB3
    pt = (l0 + lax.broadcasted_iota(jnp.int32, (_LB3, _KEEP), 0) == ll_ref[0]
          ).astype(jnp.bfloat16)
    sp_ref[0] = (
        lax.dot_general(ot0_ref[...], pt, (((1,), (1,)), ((), ())),
                        preferred_element_type=jnp.float32)
        + lax.dot_general(ot1_ref[...], pt, (((1,), (1,)), ((), ())),
                          preferred_element_type=jnp.float32))
    y_ref[0] = (
        lax.dot_general(yg0_ref[...], pt, (((1,), (1,)), ((), ())),
                        preferred_element_type=jnp.float32)
        + lax.dot_general(yg1_ref[...], pt, (((1,), (1,)), ((), ())),
                          preferred_element_type=jnp.float32)
        + b2_ref[...])


def kernel(x, W1, b1, W2, b2):
    xb = x.astype(jnp.bfloat16)
    W1b = W1.astype(jnp.bfloat16)
    W2b = W2.astype(jnp.bfloat16)
    b1c = b1.reshape(_H, 1)
    b1r = b1.reshape(1, _H)
    b2c = b2.reshape(_C, 1)

    colmax = pl.pallas_call(
        _p1_body,
        grid=(_B, _L // _LB1),
        in_specs=[
            pl.BlockSpec((1, _C, _LB1), lambda b, l: (b, 0, l)),
            pl.BlockSpec((_H, _C), lambda b, l: (0, 0)),
            pl.BlockSpec((_H, 1), lambda b, l: (0, 0)),
        ],
        out_specs=pl.BlockSpec((1, 1, _LB1), lambda b, l: (b, 0, l)),
        out_shape=jax.ShapeDtypeStruct((_B, 1, _L), jnp.float32),
    )(xb, W1b, b1c)

    v, hh, ll = pl.pallas_call(
        _p2_body,
        grid=(_B, _L // _CH2),
        in_specs=[
            pl.BlockSpec((1, 1, _L), lambda b, c: (b, 0, 0)),
            pl.BlockSpec((1, _C, _CH2), lambda b, c: (b, 0, c)),
            pl.BlockSpec((_H, _C), lambda b, c: (0, 0)),
            pl.BlockSpec((1, _H), lambda b, c: (0, 0)),
        ],
        out_specs=[
            pl.BlockSpec((1, 1, _KEEP), lambda b, c: (b, 0, 0)),
            pl.BlockSpec((1, 1, _KEEP), lambda b, c: (b, 0, 0)),
            pl.BlockSpec((1, 1, _KEEP), lambda b, c: (b, 0, 0)),
        ],
        out_shape=[
            jax.ShapeDtypeStruct((_B, 1, _KEEP), jnp.float32),
            jax.ShapeDtypeStruct((_B, 1, _KEEP), jnp.int32),
            jax.ShapeDtypeStruct((_B, 1, _KEEP), jnp.int32),
        ],
        scratch_shapes=[pltpu.VMEM((_KEEP, _H), jnp.float32),
                        pltpu.VMEM((_KEEP, _C), jnp.float32),
                        pltpu.VMEM((_KEEP, 1), jnp.int32)],
    )(colmax, xb, W1b, b1r)

    sparse, y = pl.pallas_call(
        _p3_body,
        grid=(_B, _L // _LB3),
        in_specs=[
            pl.BlockSpec((1, 1, _KEEP), lambda b, l: (b, 0, 0)),
            pl.BlockSpec((1, 1, _KEEP), lambda b, l: (b, 0, 0)),
            pl.BlockSpec((1, 1, _KEEP), lambda b, l: (b, 0, 0)),
            pl.BlockSpec((_C, _H), lambda b, l: (0, 0)),
            pl.BlockSpec((_C, 1), lambda b, l: (0, 0)),
        ],
        out_specs=[
            pl.BlockSpec((1, _H, _LB3), lambda b, l: (b, 0, l)),
            pl.BlockSpec((1, _C, _LB3), lambda b, l: (b, 0, l)),
        ],
        out_shape=[
            jax.ShapeDtypeStruct((_B, _H, _L), jnp.float32),
            jax.ShapeDtypeStruct((_B, _C, _L), jnp.float32),
        ],
        scratch_shapes=[pltpu.VMEM((_H, _KEEP), jnp.bfloat16),
                        pltpu.VMEM((_H, _KEEP), jnp.bfloat16),
                        pltpu.VMEM((_C, _KEEP), jnp.bfloat16),
                        pltpu.VMEM((_C, _KEEP), jnp.bfloat16)],
    )(v, hh, ll, W2b, b2c)

    return (y, sparse)
